# baseline (speedup 1.0000x reference)
import functools

import jax
import jax.numpy as jnp
from jax import lax
from jax.experimental import pallas as pl
from jax.experimental.pallas import tpu as pltpu
from jax.experimental.pallas import tpu_sc as plsc

NC = 2
NS = 16
NW = NC * NS
L = 16


def _bytes(v):
    m = jnp.int32(0xFF)
    return (v & m,
            lax.shift_right_logical(v, jnp.int32(8)) & m,
            lax.shift_right_logical(v, jnp.int32(16)) & m,
            lax.shift_right_logical(v, jnp.int32(24)) & m)


@functools.lru_cache(maxsize=None)
def _build(N: int):
    C = 16384
    W = C // 4
    assert N % (NW * C) == 0
    PW = N // NW
    K_CH = PW // C
    NCH = NW * K_CH
    G = C // 64
    assert K_CH == 32 and NCH % L == 0 and G == 256

    mesh = plsc.VectorSubcoreMesh(
        core_axis_name="c", subcore_axis_name="s",
        num_cores=NC, num_subcores=NS,
    )
    params = pltpu.CompilerParams(needs_layout_passes=False)

    @functools.partial(
        pl.kernel,
        out_type=jax.ShapeDtypeStruct((NCH,), jnp.int32),
        mesh=mesh,
        scratch_types=[
            pltpu.VMEM((W,), jnp.int32),
            pltpu.VMEM((W,), jnp.int32),
            pltpu.VMEM((K_CH,), jnp.int32),
            pltpu.SemaphoreType.DMA,
            pltpu.SemaphoreType.DMA,
        ],
        compiler_params=params,
    )
    def count_kernel(mask_hbm, counts_hbm, mva, mvb, stage, sia, sib):
        wid = lax.axis_index("s") * NC + lax.axis_index("c")
        base = wid * (PW // 4)
        iota = lax.iota(jnp.int32, L)
        zeros = jnp.zeros((L,), jnp.int32)

        def issue(k, buf, sem):
            moff = pl.multiple_of(base + k * W, 8)
            pltpu.async_copy(mask_hbm.at[pl.ds(moff, W)], buf, sem)

        def wait(buf, sem):
            pltpu.make_async_copy(mask_hbm.at[pl.ds(0, W)], buf, sem).wait()

        def count_chunk(buf):
            tot_v = zeros
            for h in range(2):
                def vbody(j, acc):
                    return acc + buf[pl.ds(j * L, L)]
                acc = lax.fori_loop(h * (W // 32), (h + 1) * (W // 32),
                                    vbody, zeros)
                b0, b1, b2, b3 = _bytes(acc)
                tot_v = tot_v + b0 + b1 + b2 + b3
            return jnp.sum(tot_v)

        def insert(k, tot, lo, hi):
            km = k % L
            lo = jnp.where((k < L) & (iota == km), tot, lo)
            hi = jnp.where((k >= L) & (iota == km), tot, hi)
            return lo, hi

        issue(0, mva, sia)

        def body(k2, carry):
            lo, hi = carry
            k = 2 * k2
            issue(k + 1, mvb, sib)
            wait(mva, sia)
            lo, hi = insert(k, count_chunk(mva), lo, hi)
            issue(jnp.minimum(k + 2, K_CH - 1), mva, sia)
            wait(mvb, sib)
            lo, hi = insert(k + 1, count_chunk(mvb), lo, hi)
            return lo, hi

        lo, hi = lax.fori_loop(0, K_CH // 2, body, (zeros, zeros))
        wait(mva, sia)
        stage[pl.ds(0, L)] = lo
        stage[pl.ds(L, L)] = hi
        pltpu.sync_copy(stage, counts_hbm.at[pl.ds(wid * K_CH, K_CH)])

    @functools.partial(
        pl.kernel,
        out_type=jax.ShapeDtypeStruct((N,), jnp.float32),
        mesh=mesh,
        scratch_types=[
            pltpu.VMEM((W,), jnp.int32),
            pltpu.VMEM((W,), jnp.int32),
            pltpu.VMEM((C,), jnp.float32),
            pltpu.VMEM((C,), jnp.float32),
            pltpu.VMEM((C + 8,), jnp.float32),
            pltpu.VMEM((C + 8,), jnp.float32),
            pltpu.VMEM((NCH,), jnp.int32),
            pltpu.VMEM((NCH,), jnp.int32),
            pltpu.SemaphoreType.DMA,
            pltpu.SemaphoreType.DMA,
            pltpu.SemaphoreType.DMA,
            pltpu.SemaphoreType.DMA,
        ],
        compiler_params=params,
    )
    def scatter_kernel(mask_hbm, x_hbm, src_hbm, counts_hbm, out_hbm,
                       mva, mvb, ova, ovb, sva, svb, cvm, pvm,
                       sia, sib, soa, sob):
        wid = lax.axis_index("s") * NC + lax.axis_index("c")
        base = wid * PW
        iota = lax.iota(jnp.int32, L)
        pos0 = iota * 4

        pltpu.sync_copy(counts_hbm, cvm)

        def pbody(i, carry):
            v = cvm[pl.ds(i * L, L)]
            cs = plsc.cumsum(v)
            pvm[pl.ds(i * L, L)] = carry + cs - v
            return carry + cs[15]

        lax.fori_loop(0, NCH // L, pbody, jnp.int32(0))

        def issue_in(k, mvm, ovm, svm, sem):
            start = base + k * C
            moff = pl.multiple_of(start // 4, 8)
            pltpu.async_copy(mask_hbm.at[pl.ds(moff, W)], mvm, sem)
            pltpu.async_copy(x_hbm.at[pl.ds(start, C)], ovm, sem)
            gcid = wid * K_CH + k
            pvec = pvm[pl.ds((gcid // L) * L, L)]
            off = jnp.sum(jnp.where(iota == gcid % L, pvec, jnp.int32(0)))
            wbase = jnp.maximum(
                jnp.minimum((off // 8) * 8, jnp.int32(N - (C + 8))),
                jnp.int32(0))
            wbase = pl.multiple_of(wbase, 8)
            pltpu.async_copy(src_hbm.at[pl.ds(wbase, C + 8)], svm, sem)
            return off - wbase

        def wait_in(mvm, ovm, svm, sem):
            pltpu.make_async_copy(mask_hbm.at[pl.ds(0, W)], mvm, sem).wait()
            pltpu.make_async_copy(x_hbm.at[pl.ds(0, C)], ovm, sem).wait()
            pltpu.make_async_copy(src_hbm.at[pl.ds(0, C + 8)], svm,
                                  sem).wait()

        def issue_out(k, ovm, sem):
            start = base + k * C
            pltpu.async_copy(ovm, out_hbm.at[pl.ds(start, C)], sem)

        def wait_out(ovm, sem):
            pltpu.make_async_copy(ovm, out_hbm.at[pl.ds(base, C)], sem).wait()

        def compute(mvm, ovm, svm, delta):
            def vbody(j, carry):
                v = mvm[pl.ds(j * L, L)]
                b0, b1, b2, b3 = _bytes(v)
                s = b0 + b1 + b2 + b3
                cs = plsc.cumsum(s)
                p = (delta + carry - 1) + (cs - s)
                posb = pos0 + j * 64
                for i, b in enumerate((b0, b1, b2, b3)):
                    idx = jnp.maximum(p + b, jnp.int32(0))
                    m = b != 0
                    g = plsc.load_gather(svm, [idx], mask=m)
                    plsc.store_scatter(ovm, [posb + i], g, mask=m)
                    p = p + b
                return carry + cs[15]

            lax.fori_loop(0, G, vbody, jnp.int32(0))

        delta_a0 = issue_in(0, mva, ova, sva, sia)
        delta_b0 = issue_in(1, mvb, ovb, svb, sib)

        def body(k2, carry):
            delta_a, delta_b = carry
            k = 2 * k2
            wait_in(mva, ova, sva, sia)
            compute(mva, ova, sva, delta_a)
            issue_out(k, ova, soa)
            wait_in(mvb, ovb, svb, sib)
            compute(mvb, ovb, svb, delta_b)
            issue_out(k + 1, ovb, sob)
            wait_out(ova, soa)
            da = issue_in(jnp.minimum(k + 2, K_CH - 1), mva, ova, sva, sia)
            wait_out(ovb, sob)
            db = issue_in(jnp.minimum(k + 3, K_CH - 1), mvb, ovb, svb, sib)
            return da, db

        lax.fori_loop(0, K_CH // 2, body, (delta_a0, delta_b0))
        wait_in(mva, ova, sva, sia)
        wait_in(mvb, ovb, svb, sib)

    return count_kernel, scatter_kernel


def kernel(x, mask, source, out):
    N = x.size
    m = mask.astype(jnp.int32)
    mflat = (m[:, 0::4] + (m[:, 1::4] << 8) + (m[:, 2::4] << 16)
             + (m[:, 3::4] << 24)).reshape(-1)
    xflat = x.reshape(-1)
    sflat = source.reshape(-1)
    count_kernel, scatter_kernel = _build(N)
    counts = count_kernel(mflat)
    res = scatter_kernel(mflat, xflat, sflat, counts)
    return res.reshape(x.shape)


# speedup vs baseline: 3.7408x; 3.7408x over previous
import functools

import jax
import jax.numpy as jnp
from jax import lax
from jax.experimental import pallas as pl
from jax.experimental.pallas import tpu as pltpu
from jax.experimental.pallas import tpu_sc as plsc

NC = 2
NS = 16
NW = NC * NS
L = 16


def _bytes(v):
    m = jnp.int32(0xFF)
    return (v & m,
            lax.shift_right_logical(v, jnp.int32(8)) & m,
            lax.shift_right_logical(v, jnp.int32(16)) & m,
            lax.shift_right_logical(v, jnp.int32(24)) & m)


@functools.lru_cache(maxsize=None)
def _build(N: int):
    C = 16384
    W = C // 4
    assert N % (NW * C) == 0
    PW = N // NW
    K_CH = PW // C
    NCH = NW * K_CH
    G = C // 64
    assert K_CH == 32 and NCH % L == 0 and G == 256

    mesh = plsc.VectorSubcoreMesh(
        core_axis_name="c", subcore_axis_name="s",
        num_cores=NC, num_subcores=NS,
    )
    params = pltpu.CompilerParams(needs_layout_passes=False)

    @functools.partial(
        pl.kernel,
        out_type=jax.ShapeDtypeStruct((NCH,), jnp.int32),
        mesh=mesh,
        scratch_types=[
            pltpu.VMEM((W,), jnp.int32),
            pltpu.VMEM((W,), jnp.int32),
            pltpu.VMEM((K_CH,), jnp.int32),
            pltpu.SemaphoreType.DMA,
            pltpu.SemaphoreType.DMA,
        ],
        compiler_params=params,
    )
    def count_kernel(mask_hbm, counts_hbm, mva, mvb, stage, sia, sib):
        wid = lax.axis_index("s") * NC + lax.axis_index("c")
        base = wid * (PW // 4)
        iota = lax.iota(jnp.int32, L)
        zeros = jnp.zeros((L,), jnp.int32)

        def issue(k, buf, sem):
            moff = pl.multiple_of(base + k * W, 8)
            pltpu.async_copy(mask_hbm.at[pl.ds(moff, W)], buf, sem)

        def wait(buf, sem):
            pltpu.make_async_copy(mask_hbm.at[pl.ds(0, W)], buf, sem).wait()

        def count_chunk(buf):
            tot_v = zeros
            for h in range(2):
                def vbody(j, acc):
                    return acc + buf[pl.ds(j * L, L)]
                acc = lax.fori_loop(h * (W // 32), (h + 1) * (W // 32),
                                    vbody, zeros)
                b0, b1, b2, b3 = _bytes(acc)
                tot_v = tot_v + b0 + b1 + b2 + b3
            return jnp.sum(tot_v)

        def insert(k, tot, lo, hi):
            km = k % L
            lo = jnp.where((k < L) & (iota == km), tot, lo)
            hi = jnp.where((k >= L) & (iota == km), tot, hi)
            return lo, hi

        issue(0, mva, sia)

        def body(k2, carry):
            lo, hi = carry
            k = 2 * k2
            issue(k + 1, mvb, sib)
            wait(mva, sia)
            lo, hi = insert(k, count_chunk(mva), lo, hi)
            issue(jnp.minimum(k + 2, K_CH - 1), mva, sia)
            wait(mvb, sib)
            lo, hi = insert(k + 1, count_chunk(mvb), lo, hi)
            return lo, hi

        lo, hi = lax.fori_loop(0, K_CH // 2, body, (zeros, zeros))
        wait(mva, sia)
        stage[pl.ds(0, L)] = lo
        stage[pl.ds(L, L)] = hi
        pltpu.sync_copy(stage, counts_hbm.at[pl.ds(wid * K_CH, K_CH)])

    @functools.partial(
        pl.kernel,
        out_type=jax.ShapeDtypeStruct((N,), jnp.float32),
        mesh=mesh,
        scratch_types=[
            pltpu.VMEM((W,), jnp.int32),
            pltpu.VMEM((W,), jnp.int32),
            pltpu.VMEM((C,), jnp.float32),
            pltpu.VMEM((C,), jnp.float32),
            pltpu.VMEM((C + 8,), jnp.float32),
            pltpu.VMEM((C + 8,), jnp.float32),
            pltpu.VMEM((NCH,), jnp.int32),
            pltpu.VMEM((NCH,), jnp.int32),
            pltpu.SemaphoreType.DMA,
            pltpu.SemaphoreType.DMA,
            pltpu.SemaphoreType.DMA,
            pltpu.SemaphoreType.DMA,
        ],
        compiler_params=params,
    )
    def scatter_kernel(mask_hbm, x_hbm, src_hbm, counts_hbm, out_hbm,
                       mva, mvb, ova, ovb, sva, svb, cvm, pvm,
                       sia, sib, soa, sob):
        wid = lax.axis_index("s") * NC + lax.axis_index("c")
        base = wid * PW
        iota = lax.iota(jnp.int32, L)
        pos0 = iota * 4

        pltpu.sync_copy(counts_hbm, cvm)

        def pbody(i, carry):
            v = cvm[pl.ds(i * L, L)]
            cs = plsc.cumsum(v)
            pvm[pl.ds(i * L, L)] = carry + cs - v
            return carry + cs[15]

        lax.fori_loop(0, NCH // L, pbody, jnp.int32(0))

        def issue_in(k, mvm, ovm, svm, sem):
            start = base + k * C
            moff = pl.multiple_of(start // 4, 8)
            pltpu.async_copy(mask_hbm.at[pl.ds(moff, W)], mvm, sem)
            pltpu.async_copy(x_hbm.at[pl.ds(start, C)], ovm, sem)
            gcid = wid * K_CH + k
            pvec = pvm[pl.ds((gcid // L) * L, L)]
            off = jnp.sum(jnp.where(iota == gcid % L, pvec, jnp.int32(0)))
            wbase = jnp.maximum(
                jnp.minimum((off // 8) * 8, jnp.int32(N - (C + 8))),
                jnp.int32(0))
            wbase = pl.multiple_of(wbase, 8)
            pltpu.async_copy(src_hbm.at[pl.ds(wbase, C + 8)], svm, sem)
            return off - wbase

        def wait_in(mvm, ovm, svm, sem):
            pltpu.make_async_copy(mask_hbm.at[pl.ds(0, W)], mvm, sem).wait()
            pltpu.make_async_copy(x_hbm.at[pl.ds(0, C)], ovm, sem).wait()
            pltpu.make_async_copy(src_hbm.at[pl.ds(0, C + 8)], svm,
                                  sem).wait()

        def issue_out(k, ovm, sem):
            start = base + k * C
            pltpu.async_copy(ovm, out_hbm.at[pl.ds(start, C)], sem)

        def wait_out(ovm, sem):
            pltpu.make_async_copy(ovm, out_hbm.at[pl.ds(base, C)], sem).wait()

        def compute(mvm, ovm, svm, delta):
            def vbody(j, carry):
                v = mvm[pl.ds(j * L, L)]
                b0, b1, b2, b3 = _bytes(v)
                s = b0 + b1 + b2 + b3
                cs = plsc.cumsum(s)
                p = (delta + carry - 1) + (cs - s)
                posb = pos0 + j * 64
                for i, b in enumerate((b0, b1, b2, b3)):
                    idx = jnp.maximum(p + b, jnp.int32(0))
                    m = b != 0
                    g = plsc.load_gather(svm, [idx], mask=m)
                    plsc.store_scatter(ovm, [posb + i], g, mask=m)
                    p = p + b
                return carry + cs[15]

            lax.fori_loop(0, G, vbody, jnp.int32(0))

        delta_a0 = issue_in(0, mva, ova, sva, sia)
        delta_b0 = issue_in(1, mvb, ovb, svb, sib)

        def body(k2, carry):
            delta_a, delta_b = carry
            k = 2 * k2
            wait_in(mva, ova, sva, sia)
            compute(mva, ova, sva, delta_a)
            issue_out(k, ova, soa)
            wait_in(mvb, ovb, svb, sib)
            compute(mvb, ovb, svb, delta_b)
            issue_out(k + 1, ovb, sob)
            wait_out(ova, soa)
            da = issue_in(jnp.minimum(k + 2, K_CH - 1), mva, ova, sva, sia)
            wait_out(ovb, sob)
            db = issue_in(jnp.minimum(k + 3, K_CH - 1), mvb, ovb, svb, sib)
            return da, db

        lax.fori_loop(0, K_CH // 2, body, (delta_a0, delta_b0))
        wait_in(mva, ova, sva, sia)
        wait_in(mvb, ovb, svb, sib)

    return count_kernel, scatter_kernel


def kernel(x, mask, source, out):
    N = x.size
    mflat = mask.astype(jnp.uint8).view(jnp.int32).reshape(-1)
    xflat = x.reshape(-1)
    sflat = source.reshape(-1)
    count_kernel, scatter_kernel = _build(N)
    counts = count_kernel(mflat)
    res = scatter_kernel(mflat, xflat, sflat, counts)
    return res.reshape(x.shape)


# speedup vs baseline: 3.8044x; 1.0170x over previous
import functools

import jax
import jax.numpy as jnp
from jax import lax
from jax.experimental import pallas as pl
from jax.experimental.pallas import tpu as pltpu
from jax.experimental.pallas import tpu_sc as plsc

NC = 2
NS = 16
NW = NC * NS
L = 16


def _bytes(v):
    m = jnp.int32(0xFF)
    return (v & m,
            lax.shift_right_logical(v, jnp.int32(8)) & m,
            lax.shift_right_logical(v, jnp.int32(16)) & m,
            lax.shift_right_logical(v, jnp.int32(24)) & m)


@functools.lru_cache(maxsize=None)
def _build(N: int):
    C = 16384
    W = C // 4
    assert N % (NW * C) == 0
    PW = N // NW
    K_CH = PW // C
    NCH = NW * K_CH
    G = C // 64
    assert K_CH == 32 and NCH % L == 0 and G == 256

    mesh = plsc.VectorSubcoreMesh(
        core_axis_name="c", subcore_axis_name="s",
        num_cores=NC, num_subcores=NS,
    )
    params = pltpu.CompilerParams(needs_layout_passes=False)

    @functools.partial(
        pl.kernel,
        out_type=jax.ShapeDtypeStruct((NCH,), jnp.int32),
        mesh=mesh,
        scratch_types=[
            pltpu.VMEM((W,), jnp.int32),
            pltpu.VMEM((W,), jnp.int32),
            pltpu.VMEM((K_CH,), jnp.int32),
            pltpu.SemaphoreType.DMA,
            pltpu.SemaphoreType.DMA,
        ],
        compiler_params=params,
    )
    def count_kernel(mask_hbm, counts_hbm, mva, mvb, stage, sia, sib):
        wid = lax.axis_index("s") * NC + lax.axis_index("c")
        base = wid * (PW // 4)
        iota = lax.iota(jnp.int32, L)
        zeros = jnp.zeros((L,), jnp.int32)

        def issue(k, buf, sem):
            moff = pl.multiple_of(base + k * W, 8)
            pltpu.async_copy(mask_hbm.at[pl.ds(moff, W)], buf, sem)

        def wait(buf, sem):
            pltpu.make_async_copy(mask_hbm.at[pl.ds(0, W)], buf, sem).wait()

        def count_chunk(buf):
            tot_v = zeros
            for h in range(2):
                def vbody(j, acc):
                    return acc + buf[pl.ds(j * L, L)]
                acc = lax.fori_loop(h * (W // 32), (h + 1) * (W // 32),
                                    vbody, zeros)
                b0, b1, b2, b3 = _bytes(acc)
                tot_v = tot_v + b0 + b1 + b2 + b3
            return jnp.sum(tot_v)

        def insert(k, tot, lo, hi):
            km = k % L
            lo = jnp.where((k < L) & (iota == km), tot, lo)
            hi = jnp.where((k >= L) & (iota == km), tot, hi)
            return lo, hi

        issue(0, mva, sia)

        def body(k2, carry):
            lo, hi = carry
            k = 2 * k2
            issue(k + 1, mvb, sib)
            wait(mva, sia)
            lo, hi = insert(k, count_chunk(mva), lo, hi)
            issue(jnp.minimum(k + 2, K_CH - 1), mva, sia)
            wait(mvb, sib)
            lo, hi = insert(k + 1, count_chunk(mvb), lo, hi)
            return lo, hi

        lo, hi = lax.fori_loop(0, K_CH // 2, body, (zeros, zeros))
        wait(mva, sia)
        stage[pl.ds(0, L)] = lo
        stage[pl.ds(L, L)] = hi
        pltpu.sync_copy(stage, counts_hbm.at[pl.ds(wid * K_CH, K_CH)])

    @functools.partial(
        pl.kernel,
        out_type=jax.ShapeDtypeStruct((N,), jnp.float32),
        mesh=mesh,
        scratch_types=[
            pltpu.VMEM((W,), jnp.int32),
            pltpu.VMEM((W,), jnp.int32),
            pltpu.VMEM((C,), jnp.float32),
            pltpu.VMEM((C,), jnp.float32),
            pltpu.VMEM((C + 8,), jnp.float32),
            pltpu.VMEM((C + 8,), jnp.float32),
            pltpu.VMEM((NCH,), jnp.int32),
            pltpu.VMEM((NCH,), jnp.int32),
            pltpu.SemaphoreType.DMA,
            pltpu.SemaphoreType.DMA,
            pltpu.SemaphoreType.DMA,
            pltpu.SemaphoreType.DMA,
        ],
        compiler_params=params,
    )
    def scatter_kernel(mask_hbm, x_hbm, src_hbm, counts_hbm, out_hbm,
                       mva, mvb, ova, ovb, sva, svb, cvm, pvm,
                       sia, sib, soa, sob):
        wid = lax.axis_index("s") * NC + lax.axis_index("c")
        base = wid * PW
        iota = lax.iota(jnp.int32, L)
        pos0 = iota * 4

        pltpu.sync_copy(counts_hbm, cvm)

        def pbody(i, carry):
            v = cvm[pl.ds(i * L, L)]
            cs = plsc.cumsum(v)
            pvm[pl.ds(i * L, L)] = carry + cs - v
            return carry + cs[15]

        lax.fori_loop(0, NCH // L, pbody, jnp.int32(0))

        def issue_in(k, mvm, ovm, svm, sem):
            start = base + k * C
            moff = pl.multiple_of(start // 4, 8)
            pltpu.async_copy(mask_hbm.at[pl.ds(moff, W)], mvm, sem)
            pltpu.async_copy(x_hbm.at[pl.ds(start, C)], ovm, sem)
            gcid = wid * K_CH + k
            pvec = pvm[pl.ds((gcid // L) * L, L)]
            off = jnp.sum(jnp.where(iota == gcid % L, pvec, jnp.int32(0)))
            wbase = jnp.maximum(
                jnp.minimum((off // 8) * 8, jnp.int32(N - (C + 8))),
                jnp.int32(0))
            wbase = pl.multiple_of(wbase, 8)
            pltpu.async_copy(src_hbm.at[pl.ds(wbase, C + 8)], svm, sem)
            return off - wbase

        def wait_in(mvm, ovm, svm, sem):
            pltpu.make_async_copy(mask_hbm.at[pl.ds(0, W)], mvm, sem).wait()
            pltpu.make_async_copy(x_hbm.at[pl.ds(0, C)], ovm, sem).wait()
            pltpu.make_async_copy(src_hbm.at[pl.ds(0, C + 8)], svm,
                                  sem).wait()

        def issue_out(k, ovm, sem):
            start = base + k * C
            pltpu.async_copy(ovm, out_hbm.at[pl.ds(start, C)], sem)

        def wait_out(ovm, sem):
            pltpu.make_async_copy(ovm, out_hbm.at[pl.ds(base, C)], sem).wait()

        def compute(mvm, ovm, svm, delta):
            def vbody(j, carry):
                v = mvm[pl.ds(j * L, L)]
                b0, b1, b2, b3 = _bytes(v)
                s = b0 + b1 + b2 + b3
                cs = plsc.cumsum(s)
                p = (delta + carry - 1) + (cs - s)
                posb = pos0 + j * 64
                for i, b in enumerate((b0, b1, b2, b3)):
                    idx = jnp.maximum(p + b, jnp.int32(0))
                    m = b != 0
                    g = plsc.load_gather(svm, [idx], mask=m)
                    plsc.store_scatter(ovm, [posb + i], g, mask=m)
                    p = p + b
                return carry + cs[15]

            lax.fori_loop(0, G, vbody, jnp.int32(0), unroll=4)

        delta_a0 = issue_in(0, mva, ova, sva, sia)
        delta_b0 = issue_in(1, mvb, ovb, svb, sib)

        def body(k2, carry):
            delta_a, delta_b = carry
            k = 2 * k2
            wait_in(mva, ova, sva, sia)
            compute(mva, ova, sva, delta_a)
            issue_out(k, ova, soa)
            wait_in(mvb, ovb, svb, sib)
            compute(mvb, ovb, svb, delta_b)
            issue_out(k + 1, ovb, sob)
            wait_out(ova, soa)
            da = issue_in(jnp.minimum(k + 2, K_CH - 1), mva, ova, sva, sia)
            wait_out(ovb, sob)
            db = issue_in(jnp.minimum(k + 3, K_CH - 1), mvb, ovb, svb, sib)
            return da, db

        lax.fori_loop(0, K_CH // 2, body, (delta_a0, delta_b0))
        wait_in(mva, ova, sva, sia)
        wait_in(mvb, ovb, svb, sib)

    return count_kernel, scatter_kernel


def kernel(x, mask, source, out):
    N = x.size
    mflat = mask.astype(jnp.uint8).view(jnp.int32).reshape(-1)
    xflat = x.reshape(-1)
    sflat = source.reshape(-1)
    count_kernel, scatter_kernel = _build(N)
    counts = count_kernel(mflat)
    res = scatter_kernel(mflat, xflat, sflat, counts)
    return res.reshape(x.shape)
